# Initial kernel scaffold; baseline (speedup 1.0000x reference)
import jax
import jax.numpy as jnp
from jax.experimental import pallas as pl


def kernel(ids, table):
    raise NotImplementedError("write your pallas kernel here")


# speedup vs baseline: 5.3134x; 5.3134x over previous
import jax
import jax.numpy as jnp
from jax.experimental import pallas as pl
from jax.experimental.pallas import tpu as pltpu
from jax.experimental.pallas import tpu_sc as plsc

_WINDOW = 128


def kernel(ids, table):
    B, S = ids.shape
    V, D = table.shape
    N = B * S
    assert N % _WINDOW == 0

    ids_flat = ids.reshape(1, N).astype(jnp.int32)
    mesh = plsc.VectorSubcoreMesh(core_axis_name="c", subcore_axis_name="s")
    cp = pltpu.CompilerParams(use_tc_tiling_on_sc=False)

    @jax.jit
    def run(table_arr, idx_arr):
        @pl.kernel(
            out_type=jax.ShapeDtypeStruct((N, D), table_arr.dtype),
            mesh=mesh,
            compiler_params=cp,
        )
        def k(table_hbm, i_hbm, o_hbm):
            def body(i_vmem, o_vmem):
                pltpu.sync_copy(table_hbm.at[i_vmem.at[0]], o_vmem)

            pltpu.emit_pipeline(
                body,
                grid=(N // _WINDOW,),
                in_specs=[pl.BlockSpec((1, _WINDOW), lambda i: (0, i))],
                out_specs=[pl.BlockSpec((_WINDOW, D), lambda i: (i, 0))],
                core_axis_name=("c", "s"),
                dimension_semantics=(pltpu.PARALLEL,),
            )(i_hbm, o_hbm)

        return k(table_arr, idx_arr)

    out = run(table, ids_flat)
    return out.reshape(B, S, D)
